# baseline (speedup 1.0000x reference)
import jax
import jax.numpy as jnp
from jax.experimental import pallas as pl

_NQ = 6
_K = 1024
_C = 512
_B = 16
_T = 1024
_BT = _B * _T
_TILE = 512
_TPB = _T // _TILE
_NT = _BT // _TILE


def _trunc_bf16(v):
    u = jax.lax.bitcast_convert_type(v, jnp.uint32)
    return jax.lax.bitcast_convert_type(u & jnp.uint32(0xFFFF0000), jnp.float32)


def _rvq_kernel(x_ref, cbt_ref, hi_ref, mid_ref, lo_ref, cbn_ref,
                q_ref, idx_ref, counts_ref, stats_ref):
    t = pl.program_id(0)

    @pl.when(t == 0)
    def _init():
        counts_ref[...] = jnp.zeros_like(counts_ref)
        stats_ref[...] = jnp.zeros_like(stats_ref)

    xt = x_ref[0]
    _H = _TILE // 2
    rs = [jnp.transpose(xt[:, :_H], (1, 0)),
          jnp.transpose(xt[:, _H:], (1, 0))]
    qs = [jnp.zeros_like(rs[0]), jnp.zeros_like(rs[1])]
    iota_f = jax.lax.broadcasted_iota(jnp.int32, (_H, _K), 1).astype(jnp.float32)
    ones8 = jnp.ones((8, _H), jnp.bfloat16)
    idx_cols = [[], []]
    for i in range(_NQ):
        ssq_i = None
        cnt_i = None
        for h in range(2):
            r = rs[h]
            rn = jnp.sum(r * r, axis=1, keepdims=True)
            rd = jnp.dot(r, cbt_ref[i], preferred_element_type=jnp.float32)
            dist = (rn - 2.0 * rd) + cbn_ref[i:i + 1, :]
            mn = jnp.min(dist, axis=1, keepdims=True)
            idxf = jnp.min(jnp.where(dist == mn, iota_f, jnp.float32(_K)),
                           axis=1, keepdims=True)
            onehot = (iota_f == idxf).astype(jnp.bfloat16)
            m1 = jnp.dot(onehot, hi_ref[i], preferred_element_type=jnp.float32)
            m2 = jnp.dot(onehot, mid_ref[i], preferred_element_type=jnp.float32)
            m3 = jnp.dot(onehot, lo_ref[i], preferred_element_type=jnp.float32)
            xd = (m1 + m2) + m3
            s = xd - r
            ssq = jnp.sum(jnp.sum(s * s, axis=1, keepdims=True),
                          axis=0, keepdims=True)
            cnt8 = jnp.dot(ones8, onehot, preferred_element_type=jnp.float32)
            ssq_i = ssq if ssq_i is None else ssq_i + ssq
            cnt_i = cnt8[0:1, :] if cnt_i is None else cnt_i + cnt8[0:1, :]
            zq = r + s
            rs[h] = r - zq
            qs[h] = qs[h] + zq
            idx_cols[h].append(idxf.astype(jnp.int32))
        stats_ref[0:1, i:i + 1] += ssq_i
        counts_ref[i:i + 1, :] += cnt_i
    q_ref[0] = jnp.concatenate(
        [jnp.transpose(qs[0], (1, 0)), jnp.transpose(qs[1], (1, 0))], axis=1)
    idx_ref[...] = jnp.concatenate(
        [jnp.concatenate(idx_cols[h] + [jnp.zeros((_H, 2), jnp.int32)], axis=1)
         for h in range(2)], axis=0)

    @pl.when(t == _NT - 1)
    def _fin():
        counts = counts_ref[...]
        prob = counts / jnp.float32(_BT)
        ent = jnp.sum(prob * jnp.log(prob + 1e-7), axis=1, keepdims=True)
        perp = jnp.exp(-ent)
        rowmask = jax.lax.broadcasted_iota(jnp.int32, (8, 1), 0) < _NQ
        mean_perp = jnp.sum(jnp.where(rowmask, perp, 0.0),
                            axis=0, keepdims=True) / jnp.float32(_NQ)
        csum = jnp.sum(stats_ref[0:1, :], axis=1, keepdims=True)
        commit = csum / jnp.float32(_NQ * _BT * _C)
        stats_ref[1:2, 0:1] = commit
        stats_ref[1:2, 1:2] = mean_perp


@jax.jit
def kernel(x, codebooks):
    cbt = jnp.transpose(codebooks, (0, 2, 1))
    hi_f = _trunc_bf16(codebooks)
    mid_f = _trunc_bf16(codebooks - hi_f)
    lo_f = (codebooks - hi_f) - mid_f
    hi = hi_f.astype(jnp.bfloat16)
    mid = mid_f.astype(jnp.bfloat16)
    lo = lo_f.astype(jnp.bfloat16)
    cbn = jnp.stack([jnp.sum(codebooks[i] ** 2, axis=-1) for i in range(_NQ)])
    cbn8 = jnp.concatenate([cbn, jnp.zeros((2, _K), jnp.float32)], axis=0)

    out_shapes = [
        jax.ShapeDtypeStruct((_B, _C, _T), jnp.float32),
        jax.ShapeDtypeStruct((_BT, 8), jnp.int32),
        jax.ShapeDtypeStruct((8, _K), jnp.float32),
        jax.ShapeDtypeStruct((8, 128), jnp.float32),
    ]
    quantized, idx8, _counts, stats = pl.pallas_call(
        _rvq_kernel,
        grid=(_NT,),
        in_specs=[
            pl.BlockSpec((1, _C, _TILE), lambda t: (t // _TPB, 0, t % _TPB)),
            pl.BlockSpec((_NQ, _C, _K), lambda t: (0, 0, 0)),
            pl.BlockSpec((_NQ, _K, _C), lambda t: (0, 0, 0)),
            pl.BlockSpec((_NQ, _K, _C), lambda t: (0, 0, 0)),
            pl.BlockSpec((_NQ, _K, _C), lambda t: (0, 0, 0)),
            pl.BlockSpec((8, _K), lambda t: (0, 0)),
        ],
        out_specs=[
            pl.BlockSpec((1, _C, _TILE), lambda t: (t // _TPB, 0, t % _TPB)),
            pl.BlockSpec((_TILE, 8), lambda t: (t, 0)),
            pl.BlockSpec((8, _K), lambda t: (0, 0)),
            pl.BlockSpec((8, 128), lambda t: (0, 0)),
        ],
        out_shape=out_shapes,
    )(x, cbt, hi, mid, lo, cbn8)

    indices = jnp.transpose(idx8[:, :_NQ], (1, 0))
    return quantized, indices, stats[1, 0], stats[1, 1]


# speedup vs baseline: 1.1068x; 1.1068x over previous
import jax
import jax.numpy as jnp
from jax.experimental import pallas as pl

_NQ = 6
_K = 1024
_C = 512
_B = 16
_T = 1024
_BT = _B * _T
_TILE = 512
_TPB = _T // _TILE
_NT = _BT // _TILE


def _trunc_bf16(v):
    u = jax.lax.bitcast_convert_type(v, jnp.uint32)
    return jax.lax.bitcast_convert_type(u & jnp.uint32(0xFFFF0000), jnp.float32)


def _rvq_kernel(xf_ref, cbt_ref, hi_ref, mid_ref, lo_ref, cbn_ref,
                q_ref, idx_ref, counts_ref, stats_ref):
    t = pl.program_id(0)

    @pl.when(t == 0)
    def _init():
        counts_ref[...] = jnp.zeros_like(counts_ref)
        stats_ref[...] = jnp.zeros_like(stats_ref)

    r = jnp.transpose(xf_ref[0], (1, 0))
    q = jnp.zeros_like(r)
    iota_f = jax.lax.broadcasted_iota(jnp.int32, (_TILE, _K), 1).astype(jnp.float32)
    ones8 = jnp.ones((8, _TILE), jnp.bfloat16)
    idx_cols = []
    for i in range(_NQ):
        rn = jnp.sum(r * r, axis=1, keepdims=True)
        rd = jnp.dot(r, cbt_ref[i], preferred_element_type=jnp.float32)
        dist = (rn - 2.0 * rd) + cbn_ref[i:i + 1, :]
        mn = jnp.min(dist, axis=1, keepdims=True)
        idxf = jnp.min(jnp.where(dist == mn, iota_f, jnp.float32(_K)),
                       axis=1, keepdims=True)
        onehot = (iota_f == idxf).astype(jnp.bfloat16)
        m1 = jnp.dot(onehot, hi_ref[i], preferred_element_type=jnp.float32)
        m2 = jnp.dot(onehot, mid_ref[i], preferred_element_type=jnp.float32)
        m3 = jnp.dot(onehot, lo_ref[i], preferred_element_type=jnp.float32)
        xd = (m1 + m2) + m3
        s = xd - r
        ssq = jnp.sum(jnp.sum(s * s, axis=1, keepdims=True), axis=0, keepdims=True)
        stats_ref[0:1, i:i + 1] += ssq
        cnt8 = jnp.dot(ones8, onehot, preferred_element_type=jnp.float32)
        counts_ref[i:i + 1, :] += cnt8[0:1, :]
        zq = r + s
        r = r - zq
        q = q + zq
        idx_cols.append(idxf.astype(jnp.int32))
    q_ref[0] = jnp.transpose(q, (1, 0))
    idx_ref[...] = jnp.concatenate(
        idx_cols + [jnp.zeros((_TILE, 2), jnp.int32)], axis=1)

    @pl.when(t == _NT - 1)
    def _fin():
        counts = counts_ref[...]
        prob = counts / jnp.float32(_BT)
        ent = jnp.sum(prob * jnp.log(prob + 1e-7), axis=1, keepdims=True)
        perp = jnp.exp(-ent)
        rowmask = jax.lax.broadcasted_iota(jnp.int32, (8, 1), 0) < _NQ
        mean_perp = jnp.sum(jnp.where(rowmask, perp, 0.0),
                            axis=0, keepdims=True) / jnp.float32(_NQ)
        csum = jnp.sum(stats_ref[0:1, :], axis=1, keepdims=True)
        commit = csum / jnp.float32(_NQ * _BT * _C)
        stats_ref[1:2, 0:1] = commit
        stats_ref[1:2, 1:2] = mean_perp


@jax.jit
def kernel(x, codebooks):
    cbt = jnp.transpose(codebooks, (0, 2, 1))
    hi_f = _trunc_bf16(codebooks)
    mid_f = _trunc_bf16(codebooks - hi_f)
    lo_f = (codebooks - hi_f) - mid_f
    hi = hi_f.astype(jnp.bfloat16)
    mid = mid_f.astype(jnp.bfloat16)
    lo = lo_f.astype(jnp.bfloat16)
    cbn = jnp.stack([jnp.sum(codebooks[i] ** 2, axis=-1) for i in range(_NQ)])
    cbn8 = jnp.concatenate([cbn, jnp.zeros((2, _K), jnp.float32)], axis=0)

    out_shapes = [
        jax.ShapeDtypeStruct((_B, _C, _T), jnp.float32),
        jax.ShapeDtypeStruct((_BT, 8), jnp.int32),
        jax.ShapeDtypeStruct((8, _K), jnp.float32),
        jax.ShapeDtypeStruct((8, 128), jnp.float32),
    ]
    quantized, idx8, _counts, stats = pl.pallas_call(
        _rvq_kernel,
        grid=(_NT,),
        in_specs=[
            pl.BlockSpec((1, _C, _TILE), lambda t: (t // _TPB, 0, t % _TPB)),
            pl.BlockSpec((_NQ, _C, _K), lambda t: (0, 0, 0)),
            pl.BlockSpec((_NQ, _K, _C), lambda t: (0, 0, 0)),
            pl.BlockSpec((_NQ, _K, _C), lambda t: (0, 0, 0)),
            pl.BlockSpec((_NQ, _K, _C), lambda t: (0, 0, 0)),
            pl.BlockSpec((8, _K), lambda t: (0, 0)),
        ],
        out_specs=[
            pl.BlockSpec((1, _C, _TILE), lambda t: (t // _TPB, 0, t % _TPB)),
            pl.BlockSpec((_TILE, 8), lambda t: (t, 0)),
            pl.BlockSpec((8, _K), lambda t: (0, 0)),
            pl.BlockSpec((8, 128), lambda t: (0, 0)),
        ],
        out_shape=out_shapes,
    )(x, cbt, hi, mid, lo, cbn8)

    indices = jnp.transpose(idx8[:, :_NQ], (1, 0))
    return quantized, indices, stats[1, 0], stats[1, 1]


# speedup vs baseline: 1.1178x; 1.0099x over previous
import jax
import jax.numpy as jnp
from jax.experimental import pallas as pl

_NQ = 6
_K = 1024
_C = 512
_B = 16
_T = 1024
_BT = _B * _T
_TILE = 512
_TPB = _T // _TILE
_NT = _BT // _TILE


def _trunc_bf16(v):
    u = jax.lax.bitcast_convert_type(v, jnp.uint32)
    return jax.lax.bitcast_convert_type(u & jnp.uint32(0xFFFF0000), jnp.float32)


def _rvq_kernel(xf_ref, cbt_ref, hi_ref, mid_ref, lo_ref, cbn_ref,
                q_ref, idx_ref, counts_ref, stats_ref):
    t = pl.program_id(0)

    @pl.when(t == 0)
    def _init():
        counts_ref[...] = jnp.zeros_like(counts_ref)
        stats_ref[...] = jnp.zeros_like(stats_ref)

    r = jnp.transpose(xf_ref[0], (1, 0))
    q = jnp.zeros_like(r)
    iota_f = jax.lax.broadcasted_iota(jnp.int32, (_TILE, _K), 1).astype(jnp.float32)
    ones8 = jnp.ones((8, _TILE), jnp.bfloat16)
    idx_cols = []
    for i in range(_NQ):
        rn = jnp.sum(r * r, axis=1, keepdims=True)
        rd = jax.lax.dot_general(r, cbt_ref[i], (((1,), (1,)), ((), ())),
                                 preferred_element_type=jnp.float32)
        dist = (rn - 2.0 * rd) + cbn_ref[i:i + 1, :]
        mn = jnp.min(dist, axis=1, keepdims=True)
        idxf = jnp.min(jnp.where(dist == mn, iota_f, jnp.float32(_K)),
                       axis=1, keepdims=True)
        onehot = (iota_f == idxf).astype(jnp.bfloat16)
        m1 = jnp.dot(onehot, hi_ref[i], preferred_element_type=jnp.float32)
        m2 = jnp.dot(onehot, mid_ref[i], preferred_element_type=jnp.float32)
        m3 = jnp.dot(onehot, lo_ref[i], preferred_element_type=jnp.float32)
        xd = (m1 + m2) + m3
        s = xd - r
        ssq = jnp.sum(jnp.sum(s * s, axis=1, keepdims=True), axis=0, keepdims=True)
        stats_ref[0:1, i:i + 1] += ssq
        cnt8 = jnp.dot(ones8, onehot, preferred_element_type=jnp.float32)
        counts_ref[i:i + 1, :] += cnt8[0:1, :]
        zq = r + s
        r = r - zq
        q = q + zq
        idx_cols.append(idxf.astype(jnp.int32))
    q_ref[0] = jnp.transpose(q, (1, 0))
    idx_ref[...] = jnp.concatenate(
        idx_cols + [jnp.zeros((_TILE, 2), jnp.int32)], axis=1)

    @pl.when(t == _NT - 1)
    def _fin():
        counts = counts_ref[...]
        prob = counts / jnp.float32(_BT)
        ent = jnp.sum(prob * jnp.log(prob + 1e-7), axis=1, keepdims=True)
        perp = jnp.exp(-ent)
        rowmask = jax.lax.broadcasted_iota(jnp.int32, (8, 1), 0) < _NQ
        mean_perp = jnp.sum(jnp.where(rowmask, perp, 0.0),
                            axis=0, keepdims=True) / jnp.float32(_NQ)
        csum = jnp.sum(stats_ref[0:1, :], axis=1, keepdims=True)
        commit = csum / jnp.float32(_NQ * _BT * _C)
        stats_ref[1:2, 0:1] = commit
        stats_ref[1:2, 1:2] = mean_perp


@jax.jit
def kernel(x, codebooks):
    cbt = codebooks
    hi_f = _trunc_bf16(codebooks)
    mid_f = _trunc_bf16(codebooks - hi_f)
    lo_f = (codebooks - hi_f) - mid_f
    hi = hi_f.astype(jnp.bfloat16)
    mid = mid_f.astype(jnp.bfloat16)
    lo = lo_f.astype(jnp.bfloat16)
    cbn = jnp.stack([jnp.sum(codebooks[i] ** 2, axis=-1) for i in range(_NQ)])
    cbn8 = jnp.concatenate([cbn, jnp.zeros((2, _K), jnp.float32)], axis=0)

    out_shapes = [
        jax.ShapeDtypeStruct((_B, _C, _T), jnp.float32),
        jax.ShapeDtypeStruct((_BT, 8), jnp.int32),
        jax.ShapeDtypeStruct((8, _K), jnp.float32),
        jax.ShapeDtypeStruct((8, 128), jnp.float32),
    ]
    quantized, idx8, _counts, stats = pl.pallas_call(
        _rvq_kernel,
        grid=(_NT,),
        in_specs=[
            pl.BlockSpec((1, _C, _TILE), lambda t: (t // _TPB, 0, t % _TPB)),
            pl.BlockSpec((_NQ, _K, _C), lambda t: (0, 0, 0)),
            pl.BlockSpec((_NQ, _K, _C), lambda t: (0, 0, 0)),
            pl.BlockSpec((_NQ, _K, _C), lambda t: (0, 0, 0)),
            pl.BlockSpec((_NQ, _K, _C), lambda t: (0, 0, 0)),
            pl.BlockSpec((8, _K), lambda t: (0, 0)),
        ],
        out_specs=[
            pl.BlockSpec((1, _C, _TILE), lambda t: (t // _TPB, 0, t % _TPB)),
            pl.BlockSpec((_TILE, 8), lambda t: (t, 0)),
            pl.BlockSpec((8, _K), lambda t: (0, 0)),
            pl.BlockSpec((8, 128), lambda t: (0, 0)),
        ],
        out_shape=out_shapes,
    )(x, cbt, hi, mid, lo, cbn8)

    indices = jnp.transpose(idx8[:, :_NQ], (1, 0))
    return quantized, indices, stats[1, 0], stats[1, 1]


# speedup vs baseline: 1.2128x; 1.0850x over previous
import jax
import jax.numpy as jnp
from jax.experimental import pallas as pl

_NQ = 6
_K = 1024
_C = 512
_B = 16
_T = 1024
_BT = _B * _T
_TILE = 1024
_TPB = _T // _TILE
_NT = _BT // _TILE


def _trunc_bf16(v):
    u = jax.lax.bitcast_convert_type(v, jnp.uint32)
    return jax.lax.bitcast_convert_type(u & jnp.uint32(0xFFFF0000), jnp.float32)


def _rvq_kernel(xf_ref, cbt_ref, hi_ref, mid_ref, lo_ref, cbn_ref,
                q_ref, idx_ref, counts_ref, stats_ref):
    t = pl.program_id(0)

    @pl.when(t == 0)
    def _init():
        counts_ref[...] = jnp.zeros_like(counts_ref)
        stats_ref[...] = jnp.zeros_like(stats_ref)

    r = jnp.transpose(xf_ref[0], (1, 0))
    q = jnp.zeros_like(r)
    iota_f = jax.lax.broadcasted_iota(jnp.int32, (_TILE, _K), 1).astype(jnp.float32)
    ones8 = jnp.ones((8, _TILE), jnp.bfloat16)
    idx_cols = []
    for i in range(_NQ):
        rn = jnp.sum(r * r, axis=1, keepdims=True)
        rd = jax.lax.dot_general(r, cbt_ref[i], (((1,), (1,)), ((), ())),
                                 preferred_element_type=jnp.float32)
        dist = (rn - 2.0 * rd) + cbn_ref[i:i + 1, :]
        mn = jnp.min(dist, axis=1, keepdims=True)
        idxf = jnp.min(jnp.where(dist == mn, iota_f, jnp.float32(_K)),
                       axis=1, keepdims=True)
        onehot = (iota_f == idxf).astype(jnp.bfloat16)
        m1 = jnp.dot(onehot, hi_ref[i], preferred_element_type=jnp.float32)
        m2 = jnp.dot(onehot, mid_ref[i], preferred_element_type=jnp.float32)
        m3 = jnp.dot(onehot, lo_ref[i], preferred_element_type=jnp.float32)
        xd = (m1 + m2) + m3
        s = xd - r
        ssq = jnp.sum(jnp.sum(s * s, axis=1, keepdims=True), axis=0, keepdims=True)
        stats_ref[0:1, i:i + 1] += ssq
        cnt8 = jnp.dot(ones8, onehot, preferred_element_type=jnp.float32)
        counts_ref[i:i + 1, :] += cnt8[0:1, :]
        zq = r + s
        r = r - zq
        q = q + zq
        idx_cols.append(idxf.astype(jnp.int32))
    q_ref[0] = jnp.transpose(q, (1, 0))
    idx_ref[...] = jnp.concatenate(
        idx_cols + [jnp.zeros((_TILE, 2), jnp.int32)], axis=1)

    @pl.when(t == _NT - 1)
    def _fin():
        counts = counts_ref[...]
        prob = counts / jnp.float32(_BT)
        ent = jnp.sum(prob * jnp.log(prob + 1e-7), axis=1, keepdims=True)
        perp = jnp.exp(-ent)
        rowmask = jax.lax.broadcasted_iota(jnp.int32, (8, 1), 0) < _NQ
        mean_perp = jnp.sum(jnp.where(rowmask, perp, 0.0),
                            axis=0, keepdims=True) / jnp.float32(_NQ)
        csum = jnp.sum(stats_ref[0:1, :], axis=1, keepdims=True)
        commit = csum / jnp.float32(_NQ * _BT * _C)
        stats_ref[1:2, 0:1] = commit
        stats_ref[1:2, 1:2] = mean_perp


@jax.jit
def kernel(x, codebooks):
    cbt = codebooks
    hi_f = _trunc_bf16(codebooks)
    mid_f = _trunc_bf16(codebooks - hi_f)
    lo_f = (codebooks - hi_f) - mid_f
    hi = hi_f.astype(jnp.bfloat16)
    mid = mid_f.astype(jnp.bfloat16)
    lo = lo_f.astype(jnp.bfloat16)
    cbn = jnp.stack([jnp.sum(codebooks[i] ** 2, axis=-1) for i in range(_NQ)])
    cbn8 = jnp.concatenate([cbn, jnp.zeros((2, _K), jnp.float32)], axis=0)

    out_shapes = [
        jax.ShapeDtypeStruct((_B, _C, _T), jnp.float32),
        jax.ShapeDtypeStruct((_BT, 8), jnp.int32),
        jax.ShapeDtypeStruct((8, _K), jnp.float32),
        jax.ShapeDtypeStruct((8, 128), jnp.float32),
    ]
    quantized, idx8, _counts, stats = pl.pallas_call(
        _rvq_kernel,
        grid=(_NT,),
        in_specs=[
            pl.BlockSpec((1, _C, _TILE), lambda t: (t // _TPB, 0, t % _TPB)),
            pl.BlockSpec((_NQ, _K, _C), lambda t: (0, 0, 0)),
            pl.BlockSpec((_NQ, _K, _C), lambda t: (0, 0, 0)),
            pl.BlockSpec((_NQ, _K, _C), lambda t: (0, 0, 0)),
            pl.BlockSpec((_NQ, _K, _C), lambda t: (0, 0, 0)),
            pl.BlockSpec((8, _K), lambda t: (0, 0)),
        ],
        out_specs=[
            pl.BlockSpec((1, _C, _TILE), lambda t: (t // _TPB, 0, t % _TPB)),
            pl.BlockSpec((_TILE, 8), lambda t: (t, 0)),
            pl.BlockSpec((8, _K), lambda t: (0, 0)),
            pl.BlockSpec((8, 128), lambda t: (0, 0)),
        ],
        out_shape=out_shapes,
    )(x, cbt, hi, mid, lo, cbn8)

    indices = jnp.transpose(idx8[:, :_NQ], (1, 0))
    return quantized, indices, stats[1, 0], stats[1, 1]
